# Initial kernel scaffold; baseline (speedup 1.0000x reference)
import jax
import jax.numpy as jnp
from jax.experimental import pallas as pl


def kernel(inputs, embedding_weight):
    raise NotImplementedError("write your pallas kernel here")


# speedup vs baseline: 1.4463x; 1.4463x over previous
import functools

import jax
import jax.numpy as jnp
from jax.experimental import pallas as pl
from jax.experimental.pallas import tpu as pltpu

EMB_DIM = 64
NUM_EMB = 1024
COMMIT = 0.25
ROWS = 16384
TILE = 1024
GRID = ROWS // TILE


def _vq_kernel(x_ref, e_ref, enc_ref, q_ref, loss_ref, perp_ref,
               counts_ref, loss_acc_ref):
    i = pl.program_id(0)
    x = x_ref[...]
    e = e_ref[...]

    x2 = jnp.sum(x * x, axis=1, keepdims=True)
    e2 = jnp.sum(e * e, axis=1)
    xe = jax.lax.dot_general(
        x, e, (((1,), (1,)), ((), ())),
        preferred_element_type=jnp.float32)
    d = x2 + e2[None, :] - 2.0 * xe

    dmin = jnp.min(d, axis=1, keepdims=True)
    col = jax.lax.broadcasted_iota(jnp.int32, d.shape, 1)
    idx = jnp.min(jnp.where(d == dmin, col, NUM_EMB), axis=1, keepdims=True)

    onehot = (col == idx).astype(jnp.float32)
    enc_ref[...] = onehot

    q = jax.lax.dot_general(
        onehot, e, (((1,), (0,)), ((), ())),
        preferred_element_type=jnp.float32)
    q_ref[...] = x + (q - x)

    @pl.when(i == 0)
    def _init():
        loss_acc_ref[0, 0] = 0.0
        counts_ref[...] = jnp.zeros_like(counts_ref)

    diff = q - x
    loss_acc_ref[0, 0] += jnp.sum(diff * diff)
    counts_ref[...] += jnp.sum(onehot, axis=0, keepdims=True)

    @pl.when(i == GRID - 1)
    def _fini():
        loss_ref[0, 0] = COMMIT * loss_acc_ref[0, 0] / (ROWS * EMB_DIM)
        p = counts_ref[...] / ROWS
        perp_ref[0, 0] = jnp.exp(-jnp.sum(p * jnp.log(p + 1e-10)))


@functools.partial(jax.jit)
def kernel(inputs, embedding_weight):
    x = jnp.transpose(inputs, (0, 2, 3, 1))
    flat = x.reshape(ROWS, EMB_DIM)

    enc, qflat, loss, perp = pl.pallas_call(
        _vq_kernel,
        grid=(GRID,),
        in_specs=[
            pl.BlockSpec((TILE, EMB_DIM), lambda i: (i, 0)),
            pl.BlockSpec((NUM_EMB, EMB_DIM), lambda i: (0, 0)),
        ],
        out_specs=[
            pl.BlockSpec((TILE, NUM_EMB), lambda i: (i, 0)),
            pl.BlockSpec((TILE, EMB_DIM), lambda i: (i, 0)),
            pl.BlockSpec((1, 1), lambda i: (0, 0), memory_space=pltpu.SMEM),
            pl.BlockSpec((1, 1), lambda i: (0, 0), memory_space=pltpu.SMEM),
        ],
        out_shape=[
            jax.ShapeDtypeStruct((ROWS, NUM_EMB), jnp.float32),
            jax.ShapeDtypeStruct((ROWS, EMB_DIM), jnp.float32),
            jax.ShapeDtypeStruct((1, 1), jnp.float32),
            jax.ShapeDtypeStruct((1, 1), jnp.float32),
        ],
        scratch_shapes=[
            pltpu.VMEM((1, NUM_EMB), jnp.float32),
            pltpu.SMEM((1, 1), jnp.float32),
        ],
    )(flat, embedding_weight)

    quantized = qflat.reshape(x.shape)
    quantized_out = jnp.transpose(quantized, (0, 3, 1, 2))
    return (loss[0, 0], quantized_out, perp[0, 0], enc)
